# baseline (speedup 1.0000x reference)
import functools

import jax
import jax.numpy as jnp
from jax.experimental import pallas as pl
from jax.experimental.pallas import tpu as pltpu

_NATTRS = 115
_NOBJS = 245
_NN = _NATTRS + _NOBJS
_NE = 1262
_NPAIRS = _NATTRS * _NOBJS
_BATCH = 1024
_TILE = 1024
_HID1 = 1000
_HID2 = 800

_F32 = jnp.float32
_BF16 = jnp.bfloat16


def _ln(x, g, b, eps=1e-5):
    m = jnp.mean(x, axis=-1, keepdims=True)
    v = jnp.mean((x - m) * (x - m), axis=-1, keepdims=True)
    return (x - m) * jax.lax.rsqrt(v + eps) * g + b


def _dot(a, b):
    return jnp.dot(a, b, preferred_element_type=_F32)


def _graph_body(nodes_ref, edge_ref, wl1_ref, bl1_ref, wr1_ref,
                wl2_ref, bl2_ref, wr2_ref, w1a_ref, w1b_ref, pb1_ref,
                at_ref, ot_ref, m_ref, r_ref):
    nodes = nodes_ref[...]
    src = edge_ref[0, :]
    dst = edge_ref[1, :]
    row = jax.lax.broadcasted_iota(jnp.int32, (_NN, _NE), 0)
    doh = (row == dst[None, :]).astype(_F32)
    soh = (row == src[None, :]).astype(_F32)
    m = jax.lax.dot_general(doh, soh, (((1,), (1,)), ((), ())),
                            preferred_element_type=_F32)
    cnt = jnp.sum(doh, axis=1)
    inv = 1.0 / jnp.maximum(cnt, 1.0)

    mean1 = _dot(m, nodes) * inv[:, None]
    h = jnp.maximum(_dot(mean1, wl1_ref[...]) + bl1_ref[...]
                    + _dot(nodes, wr1_ref[...]), 0.0)
    mean2 = _dot(m, h) * inv[:, None]
    enc = (_dot(mean2, wl2_ref[...]) + bl2_ref[...] + _dot(h, wr2_ref[...]))

    at = jax.lax.dot_general(w1a_ref[...], enc[:_NATTRS],
                             (((0,), (1,)), ((), ())),
                             preferred_element_type=_F32) + pb1_ref[...].T
    ot = jax.lax.dot_general(w1b_ref[...], enc[_NATTRS:],
                             (((0,), (1,)), ((), ())),
                             preferred_element_type=_F32)
    at_ref[...] = at
    ot_ref[...] = ot

    s_a = jnp.sum(at, axis=0)[:, None]
    s_o = jnp.sum(ot, axis=0)[None, :]
    sq_a = jnp.sum(at * at, axis=0)[:, None]
    sq_o = jnp.sum(ot * ot, axis=0)[None, :]
    cross = jax.lax.dot_general(at, ot, (((0,), (0,)), ((), ())),
                                preferred_element_type=_F32)
    mean_p = (s_a + s_o) * (1.0 / _HID1)
    var_p = (sq_a + 2.0 * cross + sq_o) * (1.0 / _HID1) - mean_p * mean_p
    m_ref[...] = mean_p
    r_ref[...] = jax.lax.rsqrt(var_p + 1e-5)


def _img_body(x_ref, w1_ref, b1_ref, g1_ref, be1_ref,
              w2_ref, b2_ref, g2_ref, be2_ref,
              w3_ref, b3_ref, g3_ref, be3_ref, out_ref):
    i = jnp.maximum(_ln(_dot(x_ref[...], w1_ref[...]) + b1_ref[...],
                        g1_ref[...], be1_ref[...]), 0.0)
    i = jnp.maximum(_ln(_dot(i, w2_ref[...]) + b2_ref[...],
                        g2_ref[...], be2_ref[...]), 0.0)
    out_ref[...] = _ln(_dot(i, w3_ref[...]) + b3_ref[...],
                       g3_ref[...], be3_ref[...])


def _pair_body(at_ref, ot_ref, m_ref, r_ref, g1_ref, be1_ref,
               w2x_ref, b2x_ref, g2_ref, be2_ref, img_ref, out_ref):
    t = pl.program_id(0)
    c = t * _TILE + jax.lax.broadcasted_iota(jnp.int32, (1, _TILE), 1)
    a_iota = jax.lax.broadcasted_iota(jnp.int32, (_NATTRS + 1, 1), 0)
    ge = c >= a_iota * _NOBJS
    oh_i = jnp.logical_and(ge[:_NATTRS, :],
                           jnp.logical_not(ge[1:, :])).astype(_BF16)
    i_idx = jnp.sum(ge[1:, :].astype(jnp.int32), axis=0, keepdims=True)
    j_idx = c - _NOBJS * i_idx
    j_iota = jax.lax.broadcasted_iota(jnp.int32, (_NOBJS, 1), 0)
    oh_j = (j_idx == j_iota).astype(_BF16)

    pre = _dot(at_ref[...], oh_i) + _dot(ot_ref[...], oh_j)
    q = jnp.maximum((pre - m_ref[...]) * r_ref[...] * g1_ref[...]
                    + be1_ref[...], 0.0)
    zx = _dot(w2x_ref[...], q.astype(_BF16)) + b2x_ref[...]
    z = zx[:_HID2, :]
    m2 = zx[_HID2:, :]
    s2 = jnp.mean(z * z, axis=0, keepdims=True)
    r2 = jax.lax.rsqrt(s2 - m2 * m2 + 1e-5)
    ap = (z - m2) * r2 * g2_ref[...] + be2_ref[...]
    out_ref[...] = _dot(img_ref[...], ap.astype(_BF16))


def _full(shape):
    return pl.BlockSpec(shape, lambda *_: tuple(0 for _ in shape))


def kernel(x_img, nodes, params, edge_index):
    p = params
    r = lambda v: v.reshape(1, -1)
    col = lambda v: v.reshape(-1, 1)

    at_mat, ot_mat, m_pair, r_pair = pl.pallas_call(
        _graph_body,
        out_shape=(jax.ShapeDtypeStruct((_HID1, _NATTRS), _F32),
                   jax.ShapeDtypeStruct((_HID1, _NOBJS), _F32),
                   jax.ShapeDtypeStruct((_NATTRS, _NOBJS), _F32),
                   jax.ShapeDtypeStruct((_NATTRS, _NOBJS), _F32)),
    )(nodes, edge_index,
      p['sWl1'], r(p['sbl1']), p['sWr1'],
      p['sWl2'], r(p['sbl2']), p['sWr2'],
      p['pW1'][:512], p['pW1'][512:], r(p['pb1']))

    img_feats = pl.pallas_call(
        _img_body,
        out_shape=jax.ShapeDtypeStruct((_BATCH, _HID2), _F32),
    )(x_img, p['iW1'], r(p['ib1']), r(p['ig1']), r(p['ibe1']),
      p['iW2'], r(p['ib2']), r(p['ig2']), r(p['ibe2']),
      p['iW3'], r(p['ib3']), r(p['ig3']), r(p['ibe3']))

    w2t = p['pW2'].T
    w2bar = jnp.mean(p['pW2'], axis=1)[None, :]
    w2x = jnp.concatenate([w2t, w2bar], axis=0)
    b2x = jnp.concatenate([p['pb2'], jnp.mean(p['pb2'])[None]])[:, None]

    grid = (pl.cdiv(_NPAIRS, _TILE),)
    pred = pl.pallas_call(
        _pair_body,
        grid=grid,
        in_specs=[
            _full((_HID1, _NATTRS)), _full((_HID1, _NOBJS)),
            pl.BlockSpec((1, _TILE), lambda t: (0, t)),
            pl.BlockSpec((1, _TILE), lambda t: (0, t)),
            _full((_HID1, 1)), _full((_HID1, 1)),
            _full((_HID2 + 1, _HID1)), _full((_HID2 + 1, 1)),
            _full((_HID2, 1)), _full((_HID2, 1)),
            _full((_BATCH, _HID2)),
        ],
        out_specs=pl.BlockSpec((_BATCH, _TILE), lambda t: (0, t)),
        out_shape=jax.ShapeDtypeStruct((_BATCH, _NPAIRS), _F32),
    )(at_mat.astype(_BF16), ot_mat.astype(_BF16),
      m_pair.reshape(1, _NPAIRS), r_pair.reshape(1, _NPAIRS),
      col(p['pg1']), col(p['pbe1']),
      w2x.astype(_BF16), b2x,
      col(p['pg2']), col(p['pbe2']), img_feats.astype(_BF16))

    return pred
